# Initial kernel scaffold; baseline (speedup 1.0000x reference)
import jax
import jax.numpy as jnp
from jax.experimental import pallas as pl


def kernel(x, edge_index, edge_weight, W0_xz, W1_xz, b_xz, W0_hz, W1_hz, b_hz, W0_xr, W1_xr, b_xr, W0_hr, W1_hr, b_hr, W0_xh, W1_xh, b_xh, W0_hh, W1_hh, b_hh, W_lin, b_lin):
    raise NotImplementedError("write your pallas kernel here")


# speedup vs baseline: 13.8868x; 13.8868x over previous
import functools

import jax
import jax.numpy as jnp
from jax import lax
from jax.experimental import pallas as pl
from jax.experimental.pallas import tpu as pltpu
from jax.experimental.pallas import tpu_sc as plsc

_N = 10000
_E = 320000
_F = 128
_B = 80
_NC = 2
_NS = 16
_ROW_CHUNKS = _N // _B
_E_PER_TILE_DEG = _E // _NS
_DEG_CHUNKS = _E_PER_TILE_DEG // _B
_E_PER_TILE = _E // (_NC * _NS)
_EDGE_CHUNKS = _E_PER_TILE // _B


def _rsqrt16(d):
    i = lax.bitcast_convert_type(d, jnp.int32)
    i = jnp.int32(0x5F3759DF) - (i >> 1)
    r = lax.bitcast_convert_type(i, jnp.float32)
    for _ in range(4):
        r = r * (1.5 - 0.5 * d * r * r)
    return jnp.where(d > 0.0, r, 0.0)


_sc_mesh = plsc.VectorSubcoreMesh(core_axis_name="c", subcore_axis_name="s")


@functools.partial(
    pl.kernel,
    out_type=jax.ShapeDtypeStruct((_NC * _N, _F), jnp.float32),
    mesh=_sc_mesh,
    compiler_params=pltpu.CompilerParams(needs_layout_passes=False),
    scratch_types=[
        pltpu.VMEM((_B,), jnp.int32),
        pltpu.VMEM((_B,), jnp.int32),
        pltpu.VMEM((_B,), jnp.float32),
        pltpu.VMEM((_B,), jnp.float32),
        pltpu.VMEM((_B, _F), jnp.float32),
        pltpu.VMEM((_N,), jnp.float32),
        pltpu.VMEM_SHARED((_N, _F), jnp.float32),
        pltpu.VMEM_SHARED((_N,), jnp.float32),
        pltpu.VMEM_SHARED((_N,), jnp.float32),
        pltpu.SemaphoreType.DMA,
    ],
)
def _sc_sparse(x_hbm, src_hbm, dst_hbm, ew_hbm, out_hbm,
               src_v, dst_v, ew_v, lhat_v, rows_v, dis_v,
               tmp_sh, deg_sh, dis_sh, sem):
    cc = lax.axis_index("c")
    ss = lax.axis_index("s")

    for g in range(_B // 16):
        lhat_v[pl.ds(g * 16, 16)] = jnp.zeros((16,), jnp.float32)

    def _zero_rows(r, carry):
        for j in range(_F // 16):
            rows_v[r, pl.ds(j * 16, 16)] = jnp.zeros((16,), jnp.float32)
        return carry

    lax.fori_loop(0, _B, _zero_rows, 0)

    for j in range(8):
        ci = ss + j * _NS

        @pl.when(ci < _ROW_CHUNKS)
        def _():
            start = ci * _B
            pltpu.sync_copy(rows_v, tmp_sh.at[pl.ds(start, _B)])
            pltpu.sync_copy(lhat_v, deg_sh.at[pl.ds(start, _B)])

    plsc.subcore_barrier()

    def _deg_chunk(k, carry):
        base = ss * _E_PER_TILE_DEG + k * _B
        pltpu.sync_copy(src_hbm.at[pl.ds(base, _B)], src_v)
        pltpu.sync_copy(ew_hbm.at[pl.ds(base, _B)], ew_v)
        pltpu.sync_copy(ew_v, deg_sh.at[src_v], add=True)
        return carry

    lax.fori_loop(0, _DEG_CHUNKS, _deg_chunk, 0)
    plsc.subcore_barrier()

    for j in range(8):
        ci = ss + j * _NS

        @pl.when(ci < _ROW_CHUNKS)
        def _():
            start = ci * _B
            pltpu.sync_copy(deg_sh.at[pl.ds(start, _B)], ew_v)
            for g in range(_B // 16):
                lhat_v[pl.ds(g * 16, 16)] = _rsqrt16(ew_v[pl.ds(g * 16, 16)])
            pltpu.sync_copy(lhat_v, dis_sh.at[pl.ds(start, _B)])

    plsc.subcore_barrier()

    pltpu.sync_copy(dis_sh, dis_v)

    def _edge_chunk(k, carry):
        base = cc * (_E // _NC) + ss * _E_PER_TILE + k * _B
        pltpu.sync_copy(src_hbm.at[pl.ds(base, _B)], src_v)
        pltpu.sync_copy(dst_hbm.at[pl.ds(base, _B)], dst_v)
        pltpu.sync_copy(ew_hbm.at[pl.ds(base, _B)], ew_v)
        pltpu.async_copy(x_hbm.at[src_v], rows_v, sem).wait()
        for g in range(_B // 16):
            sv = src_v[pl.ds(g * 16, 16)]
            dv = dst_v[pl.ds(g * 16, 16)]
            dqs = plsc.load_gather(dis_v, [sv])
            dqd = plsc.load_gather(dis_v, [dv])
            lhat_v[pl.ds(g * 16, 16)] = -(dqs * ew_v[pl.ds(g * 16, 16)] * dqd)

        def _scale_grp(g, carry2):
            lv = lhat_v[pl.ds(g * 16, 16)]
            for i in range(16):
                s = lv[i]
                r = g * 16 + i
                for jj in range(_F // 16):
                    rows_v[r, pl.ds(jj * 16, 16)] = (
                        rows_v[r, pl.ds(jj * 16, 16)] * s)
            return carry2

        lax.fori_loop(0, _B // 16, _scale_grp, 0)
        pltpu.sync_copy(rows_v, tmp_sh.at[dst_v], add=True)
        return carry

    lax.fori_loop(0, _EDGE_CHUNKS, _edge_chunk, 0)
    plsc.subcore_barrier()

    for j in range(8):
        ci = ss + j * _NS

        @pl.when(ci < _ROW_CHUNKS)
        def _():
            start = ci * _B
            pltpu.sync_copy(tmp_sh.at[pl.ds(start, _B)],
                            out_hbm.at[pl.ds(cc * _N + start, _B)])


_R = 2000


def _tc_body(x_ref, t0_ref, t1_ref, w0_ref, w1_ref, bc_ref, wl_ref, bl_ref,
             out_ref):
    tx = t0_ref[...] + t1_ref[...]
    a = (jnp.dot(x_ref[...], w0_ref[...], preferred_element_type=jnp.float32)
         + jnp.dot(tx, w1_ref[...], preferred_element_type=jnp.float32)
         + bc_ref[...])
    z = jax.nn.sigmoid(a[:, :_F])
    ht = jnp.tanh(a[:, _F:])
    h = (1.0 - z) * ht
    out_ref[...] = (jnp.dot(jnp.maximum(h, 0.0), wl_ref[...],
                            preferred_element_type=jnp.float32) + bl_ref[...])


_tc_dense = pl.pallas_call(
    _tc_body,
    grid=(_N // _R,),
    in_specs=[
        pl.BlockSpec((_R, _F), lambda i: (i, 0)),
        pl.BlockSpec((_R, _F), lambda i: (i, 0)),
        pl.BlockSpec((_R, _F), lambda i: (i, 0)),
        pl.BlockSpec((_F, 2 * _F), lambda i: (0, 0)),
        pl.BlockSpec((_F, 2 * _F), lambda i: (0, 0)),
        pl.BlockSpec((1, 2 * _F), lambda i: (0, 0)),
        pl.BlockSpec((_F, 1), lambda i: (0, 0)),
        pl.BlockSpec((1, 1), lambda i: (0, 0)),
    ],
    out_specs=pl.BlockSpec((_R, 1), lambda i: (i, 0)),
    out_shape=jax.ShapeDtypeStruct((_N, 1), jnp.float32),
)


def kernel(x, edge_index, edge_weight, W0_xz, W1_xz, b_xz, W0_hz, W1_hz, b_hz,
           W0_xr, W1_xr, b_xr, W0_hr, W1_hr, b_hr, W0_xh, W1_xh, b_xh,
           W0_hh, W1_hh, b_hh, W_lin, b_lin):
    tmp = _sc_sparse(x, edge_index[0], edge_index[1], edge_weight)
    W0c = jnp.concatenate([W0_xz, W0_xh], axis=1)
    W1c = jnp.concatenate([W1_xz, W1_xh], axis=1)
    bc = jnp.concatenate([b_xz + b_hz, b_xh + b_hh]).reshape(1, 2 * _F)
    return _tc_dense(x, tmp[:_N], tmp[_N:], W0c, W1c, bc, W_lin,
                     b_lin.reshape(1, 1))


# speedup vs baseline: 31.8777x; 2.2955x over previous
import functools

import jax
import jax.numpy as jnp
from jax import lax
from jax.experimental import pallas as pl
from jax.experimental.pallas import tpu as pltpu
from jax.experimental.pallas import tpu_sc as plsc

_N = 10000
_E = 320000
_F = 128
_B = 80
_NC = 2
_NS = 16
_ROW_CHUNKS = _N // _B
_E_PER_TILE_DEG = _E // _NS
_DEG_CHUNKS = _E_PER_TILE_DEG // _B
_E_PER_TILE = _E // (_NC * _NS)
_EDGE_CHUNKS = _E_PER_TILE // _B


def _rsqrt16(d):
    i = lax.bitcast_convert_type(d, jnp.int32)
    i = jnp.int32(0x5F3759DF) - (i >> 1)
    r = lax.bitcast_convert_type(i, jnp.float32)
    for _ in range(4):
        r = r * (1.5 - 0.5 * d * r * r)
    return jnp.where(d > 0.0, r, 0.0)


_sc_mesh = plsc.VectorSubcoreMesh(core_axis_name="c", subcore_axis_name="s")


@functools.partial(
    pl.kernel,
    out_type=jax.ShapeDtypeStruct((_NC * _N, _F), jnp.float32),
    mesh=_sc_mesh,
    compiler_params=pltpu.CompilerParams(needs_layout_passes=False),
    scratch_types=[
        pltpu.VMEM((_B,), jnp.int32),
        pltpu.VMEM((_B,), jnp.int32),
        pltpu.VMEM((_B,), jnp.int32),
        pltpu.VMEM((_B,), jnp.int32),
        pltpu.VMEM((_B,), jnp.float32),
        pltpu.VMEM((_B,), jnp.float32),
        pltpu.VMEM((_B, _F), jnp.float32),
        pltpu.VMEM((_B, _F), jnp.float32),
        pltpu.VMEM((_N,), jnp.float32),
        pltpu.VMEM_SHARED((_N, _F), jnp.float32),
        pltpu.VMEM_SHARED((_N,), jnp.float32),
        pltpu.VMEM_SHARED((_N,), jnp.float32),
        pltpu.SemaphoreType.DMA,
        pltpu.SemaphoreType.DMA,
        pltpu.SemaphoreType.DMA,
        pltpu.SemaphoreType.DMA,
    ],
)
def _sc_sparse(x_hbm, src_hbm, dst_hbm, ew_hbm, out_hbm,
               src0, src1, dst0, dst1, ew0, ew1, rows0, rows1, dis_v,
               tmp_sh, deg_sh, dis_sh, semA0, semA1, semG0, semG1):
    cc = lax.axis_index("c")
    ss = lax.axis_index("s")

    def issue_idx(base, srcb, dstb, ewb, sem):
        pltpu.async_copy(src_hbm.at[pl.ds(base, _B)], srcb, sem)
        pltpu.async_copy(dst_hbm.at[pl.ds(base, _B)], dstb, sem)
        pltpu.async_copy(ew_hbm.at[pl.ds(base, _B)], ewb, sem)

    def wait_idx(srcb, dstb, ewb, sem):
        pltpu.make_async_copy(src_hbm.at[pl.ds(0, _B)], srcb, sem).wait()
        pltpu.make_async_copy(dst_hbm.at[pl.ds(0, _B)], dstb, sem).wait()
        pltpu.make_async_copy(ew_hbm.at[pl.ds(0, _B)], ewb, sem).wait()

    def issue_deg(base, srcb, ewb, sem):
        pltpu.async_copy(src_hbm.at[pl.ds(base, _B)], srcb, sem)
        pltpu.async_copy(ew_hbm.at[pl.ds(base, _B)], ewb, sem)

    def wait_deg(srcb, ewb, sem):
        pltpu.make_async_copy(src_hbm.at[pl.ds(0, _B)], srcb, sem).wait()
        pltpu.make_async_copy(ew_hbm.at[pl.ds(0, _B)], ewb, sem).wait()

    def issue_gather(srcb, rowsb, sem):
        pltpu.async_copy(x_hbm.at[srcb], rowsb, sem)

    def wait_gather(srcb, rowsb, sem):
        pltpu.make_async_copy(x_hbm.at[srcb], rowsb, sem).wait()

    def scale_rows(srcb, dstb, ewb, rowsb):
        def grp(g, carry):
            sv = srcb[pl.ds(g * 16, 16)]
            dv = dstb[pl.ds(g * 16, 16)]
            lh = -(plsc.load_gather(dis_v, [sv])
                   * ewb[pl.ds(g * 16, 16)]
                   * plsc.load_gather(dis_v, [dv]))
            for i in range(16):
                s = lh[i]
                r = g * 16 + i
                for jj in range(_F // 16):
                    rowsb[r, pl.ds(jj * 16, 16)] = (
                        rowsb[r, pl.ds(jj * 16, 16)] * s)
            return carry

        lax.fori_loop(0, _B // 16, grp, 0)

    for g in range(_B // 16):
        ew0[pl.ds(g * 16, 16)] = jnp.zeros((16,), jnp.float32)

    def _zero_rows(r, carry):
        for j in range(_F // 16):
            rows0[r, pl.ds(j * 16, 16)] = jnp.zeros((16,), jnp.float32)
        return carry

    lax.fori_loop(0, _B, _zero_rows, 0)

    for j in range(8):
        ci = ss + j * _NS

        @pl.when(ci < _ROW_CHUNKS)
        def _():
            start = ci * _B
            pltpu.sync_copy(rows0, tmp_sh.at[pl.ds(start, _B)])
            pltpu.sync_copy(ew0, deg_sh.at[pl.ds(start, _B)])

    plsc.subcore_barrier()

    dbase = ss * _E_PER_TILE_DEG
    issue_deg(dbase, src0, ew0, semA0)

    def deg_pair(p, carry):
        k0 = 2 * p
        issue_deg(dbase + (k0 + 1) * _B, src1, ew1, semA1)
        wait_deg(src0, ew0, semA0)
        pltpu.sync_copy(ew0, deg_sh.at[src0], add=True)

        @pl.when(k0 + 2 < _DEG_CHUNKS)
        def _():
            issue_deg(dbase + (k0 + 2) * _B, src0, ew0, semA0)

        wait_deg(src1, ew1, semA1)
        pltpu.sync_copy(ew1, deg_sh.at[src1], add=True)
        return carry

    lax.fori_loop(0, _DEG_CHUNKS // 2, deg_pair, 0)
    plsc.subcore_barrier()

    for j in range(8):
        ci = ss + j * _NS

        @pl.when(ci < _ROW_CHUNKS)
        def _():
            start = ci * _B
            pltpu.sync_copy(deg_sh.at[pl.ds(start, _B)], ew0)
            for g in range(_B // 16):
                ew1[pl.ds(g * 16, 16)] = _rsqrt16(ew0[pl.ds(g * 16, 16)])
            pltpu.sync_copy(ew1, dis_sh.at[pl.ds(start, _B)])

    plsc.subcore_barrier()

    pltpu.sync_copy(dis_sh, dis_v)

    ebase = cc * (_E // _NC) + ss * _E_PER_TILE
    issue_idx(ebase, src0, dst0, ew0, semA0)
    wait_idx(src0, dst0, ew0, semA0)
    issue_gather(src0, rows0, semG0)
    issue_idx(ebase + _B, src1, dst1, ew1, semA1)

    def edge_pair(p, carry):
        k0 = 2 * p
        wait_idx(src1, dst1, ew1, semA1)
        issue_gather(src1, rows1, semG1)
        wait_gather(src0, rows0, semG0)
        scale_rows(src0, dst0, ew0, rows0)
        pltpu.sync_copy(rows0, tmp_sh.at[dst0], add=True)
        issue_idx(ebase + (k0 + 2) * _B, src0, dst0, ew0, semA0)
        wait_idx(src0, dst0, ew0, semA0)
        issue_gather(src0, rows0, semG0)
        wait_gather(src1, rows1, semG1)
        scale_rows(src1, dst1, ew1, rows1)
        pltpu.sync_copy(rows1, tmp_sh.at[dst1], add=True)

        @pl.when(k0 + 3 < _EDGE_CHUNKS)
        def _():
            issue_idx(ebase + (k0 + 3) * _B, src1, dst1, ew1, semA1)

        return carry

    lax.fori_loop(0, _EDGE_CHUNKS // 2, edge_pair, 0)
    wait_gather(src0, rows0, semG0)
    scale_rows(src0, dst0, ew0, rows0)
    pltpu.sync_copy(rows0, tmp_sh.at[dst0], add=True)

    plsc.subcore_barrier()

    for j in range(8):
        ci = ss + j * _NS

        @pl.when(ci < _ROW_CHUNKS)
        def _():
            start = ci * _B
            pltpu.sync_copy(tmp_sh.at[pl.ds(start, _B)],
                            out_hbm.at[pl.ds(cc * _N + start, _B)])


_R = 2000


def _tc_body(x_ref, t0_ref, t1_ref, w0_ref, w1_ref, bc_ref, wl_ref, bl_ref,
             out_ref):
    tx = t0_ref[...] + t1_ref[...]
    a = (jnp.dot(x_ref[...], w0_ref[...], preferred_element_type=jnp.float32)
         + jnp.dot(tx, w1_ref[...], preferred_element_type=jnp.float32)
         + bc_ref[...])
    z = jax.nn.sigmoid(a[:, :_F])
    ht = jnp.tanh(a[:, _F:])
    h = (1.0 - z) * ht
    out_ref[...] = (jnp.dot(jnp.maximum(h, 0.0), wl_ref[...],
                            preferred_element_type=jnp.float32) + bl_ref[...])


_tc_dense = pl.pallas_call(
    _tc_body,
    grid=(_N // _R,),
    in_specs=[
        pl.BlockSpec((_R, _F), lambda i: (i, 0)),
        pl.BlockSpec((_R, _F), lambda i: (i, 0)),
        pl.BlockSpec((_R, _F), lambda i: (i, 0)),
        pl.BlockSpec((_F, 2 * _F), lambda i: (0, 0)),
        pl.BlockSpec((_F, 2 * _F), lambda i: (0, 0)),
        pl.BlockSpec((1, 2 * _F), lambda i: (0, 0)),
        pl.BlockSpec((_F, 1), lambda i: (0, 0)),
        pl.BlockSpec((1, 1), lambda i: (0, 0)),
    ],
    out_specs=pl.BlockSpec((_R, 1), lambda i: (i, 0)),
    out_shape=jax.ShapeDtypeStruct((_N, 1), jnp.float32),
)


def kernel(x, edge_index, edge_weight, W0_xz, W1_xz, b_xz, W0_hz, W1_hz, b_hz,
           W0_xr, W1_xr, b_xr, W0_hr, W1_hr, b_hr, W0_xh, W1_xh, b_xh,
           W0_hh, W1_hh, b_hh, W_lin, b_lin):
    tmp = _sc_sparse(x, edge_index[0], edge_index[1], edge_weight)
    W0c = jnp.concatenate([W0_xz, W0_xh], axis=1)
    W1c = jnp.concatenate([W1_xz, W1_xh], axis=1)
    bc = jnp.concatenate([b_xz + b_hz, b_xh + b_hh]).reshape(1, 2 * _F)
    return _tc_dense(x, tmp[:_N], tmp[_N:], W0c, W1c, bc, W_lin,
                     b_lin.reshape(1, 1))


# speedup vs baseline: 40.8095x; 1.2802x over previous
import functools

import jax
import jax.numpy as jnp
from jax import lax
from jax.experimental import pallas as pl
from jax.experimental.pallas import tpu as pltpu
from jax.experimental.pallas import tpu_sc as plsc

_N = 10000
_E = 320000
_F = 128
_B = 128
_NC = 2
_NS = 16
_EP = 327680
_RB = 80
_ROW_CHUNKS = _N // _RB
_E_PER_TILE_DEG = _EP // _NS
_DEG_CHUNKS = _E_PER_TILE_DEG // _B
_E_PER_TILE = _EP // (_NC * _NS)
_EDGE_CHUNKS = _E_PER_TILE // _B


def _rsqrt16(d):
    i = lax.bitcast_convert_type(d, jnp.int32)
    i = jnp.int32(0x5F3759DF) - (i >> 1)
    r = lax.bitcast_convert_type(i, jnp.float32)
    for _ in range(4):
        r = r * (1.5 - 0.5 * d * r * r)
    return jnp.where(d > 0.0, r, 0.0)


_sc_mesh = plsc.VectorSubcoreMesh(core_axis_name="c", subcore_axis_name="s")


@functools.partial(
    pl.kernel,
    out_type=jax.ShapeDtypeStruct((_NC * _N, _F), jnp.float32),
    mesh=_sc_mesh,
    compiler_params=pltpu.CompilerParams(needs_layout_passes=False),
    scratch_types=(
        [pltpu.VMEM((_B,), jnp.int32) for _ in range(2)]
        + [pltpu.VMEM((_B,), jnp.int32) for _ in range(2)]
        + [pltpu.VMEM((_B,), jnp.float32) for _ in range(2)]
        + [pltpu.VMEM((_B,), jnp.int32) for _ in range(2)]
        + [pltpu.VMEM((_B,), jnp.float32) for _ in range(2)]
        + [pltpu.VMEM((_B, _F), jnp.float32) for _ in range(2)]
        + [
            pltpu.VMEM((_RB,), jnp.float32),
            pltpu.VMEM((_RB,), jnp.float32),
            pltpu.VMEM((_N,), jnp.float32),
            pltpu.VMEM_SHARED((_N, _F), jnp.float32),
            pltpu.VMEM_SHARED((_N,), jnp.float32),
            pltpu.VMEM_SHARED((_N,), jnp.float32),
        ]
        + [pltpu.SemaphoreType.DMA for _ in range(2)]
        + [pltpu.SemaphoreType.DMA for _ in range(2)]
        + [pltpu.SemaphoreType.DMA for _ in range(2)]
    ),
)
def _sc_sparse(x_hbm, src_hbm, dst_hbm, ew_hbm, out_hbm,
               src_0, src_1, dst_0, dst_1, ew_0, ew_1,
               dS_0, dS_1, eS_0, eS_1, rw_0, rw_1,
               z80a, z80b, dis_v, tmp_sh, deg_sh, dis_sh,
               sA_0, sA_1, sG_0, sG_1, sS_0, sS_1):
    src = [src_0, src_1]
    dst = [dst_0, dst_1]
    ew = [ew_0, ew_1]
    dS = [dS_0, dS_1]
    eS = [eS_0, eS_1]
    rows = [rw_0, rw_1]
    semA = [sA_0, sA_1]
    semG = [sG_0, sG_1]
    semS = [sS_0, sS_1]

    cc = lax.axis_index("c")
    ss = lax.axis_index("s")

    def issue_idx(base, s):
        pltpu.async_copy(src_hbm.at[pl.ds(base, _B)], src[s], semA[s])
        pltpu.async_copy(dst_hbm.at[pl.ds(base, _B)], dst[s], semA[s])
        pltpu.async_copy(ew_hbm.at[pl.ds(base, _B)], ew[s], semA[s])

    def wait_idx(s):
        pltpu.make_async_copy(src_hbm.at[pl.ds(0, _B)], src[s], semA[s]).wait()
        pltpu.make_async_copy(dst_hbm.at[pl.ds(0, _B)], dst[s], semA[s]).wait()
        pltpu.make_async_copy(ew_hbm.at[pl.ds(0, _B)], ew[s], semA[s]).wait()

    def issue_deg(base, s):
        pltpu.async_copy(src_hbm.at[pl.ds(base, _B)], src[s], semA[s])
        pltpu.async_copy(ew_hbm.at[pl.ds(base, _B)], ew[s], semA[s])

    def wait_deg(s):
        pltpu.make_async_copy(src_hbm.at[pl.ds(0, _B)], src[s], semA[s]).wait()
        pltpu.make_async_copy(ew_hbm.at[pl.ds(0, _B)], ew[s], semA[s]).wait()

    def issue_gather(s):
        pltpu.async_copy(x_hbm.at[src[s]], rows[s], semG[s])

    def wait_gather(s):
        pltpu.make_async_copy(x_hbm.at[src[s]], rows[s], semG[s]).wait()

    def issue_scatter(s):
        pltpu.async_copy(rows[s], tmp_sh.at[dS[s]], semS[s], add=True)

    def wait_scatter(s):
        pltpu.make_async_copy(rows[s], tmp_sh.at[dS[s]], semS[s]).wait()

    def scale_rows(s):
        def grp(g, carry):
            sv = src[s][pl.ds(g * 16, 16)]
            dv = dst[s][pl.ds(g * 16, 16)]
            lh = -(plsc.load_gather(dis_v, [sv])
                   * ew[s][pl.ds(g * 16, 16)]
                   * plsc.load_gather(dis_v, [dv]))
            for i in range(16):
                sc = lh[i]
                r = g * 16 + i
                for jj in range(_F // 16):
                    rows[s][r, pl.ds(jj * 16, 16)] = (
                        rows[s][r, pl.ds(jj * 16, 16)] * sc)
            return carry

        lax.fori_loop(0, _B // 16, grp, 0)

    for g in range(_RB // 16):
        z80a[pl.ds(g * 16, 16)] = jnp.zeros((16,), jnp.float32)

    def _zero_rows(r, carry):
        for j in range(_F // 16):
            rw_0[r, pl.ds(j * 16, 16)] = jnp.zeros((16,), jnp.float32)
        return carry

    lax.fori_loop(0, _RB, _zero_rows, 0)

    for j in range(8):
        ci = ss + j * _NS

        @pl.when(ci < _ROW_CHUNKS)
        def _():
            start = ci * _RB
            pltpu.sync_copy(rw_0.at[pl.ds(0, _RB)],
                            tmp_sh.at[pl.ds(start, _RB)])
            pltpu.sync_copy(z80a, deg_sh.at[pl.ds(start, _RB)])

    plsc.subcore_barrier()

    dbase = ss * _E_PER_TILE_DEG
    issue_deg(dbase, 0)
    issue_deg(dbase + _B, 1)

    def deg_wait_scat(s):
        pltpu.make_async_copy(eS[s], deg_sh.at[dS[s]], semS[s]).wait()

    def deg_chunk(k, s):
        wait_deg(s)
        for g in range(_B // 16):
            dS[s][pl.ds(g * 16, 16)] = src[s][pl.ds(g * 16, 16)]
            eS[s][pl.ds(g * 16, 16)] = ew[s][pl.ds(g * 16, 16)]

        @pl.when(k + 2 < _DEG_CHUNKS)
        def _():
            issue_deg(dbase + (k + 2) * _B, s)

        pltpu.async_copy(eS[s], deg_sh.at[dS[s]], semS[s], add=True)

    def deg_pair(p, carry):
        k0 = 2 * p

        @pl.when(p > 0)
        def _():
            deg_wait_scat(0)

        deg_chunk(k0, 0)

        @pl.when(p > 0)
        def _():
            deg_wait_scat(1)

        deg_chunk(k0 + 1, 1)
        return carry

    lax.fori_loop(0, _DEG_CHUNKS // 2, deg_pair, 0)
    deg_wait_scat(0)
    deg_wait_scat(1)
    plsc.subcore_barrier()

    for j in range(8):
        ci = ss + j * _NS

        @pl.when(ci < _ROW_CHUNKS)
        def _():
            start = ci * _RB
            pltpu.sync_copy(deg_sh.at[pl.ds(start, _RB)], z80a)
            for g in range(_RB // 16):
                z80b[pl.ds(g * 16, 16)] = _rsqrt16(z80a[pl.ds(g * 16, 16)])
            pltpu.sync_copy(z80b, dis_sh.at[pl.ds(start, _RB)])

    plsc.subcore_barrier()

    pltpu.sync_copy(dis_sh, dis_v)

    ebase = cc * (_EP // _NC) + ss * _E_PER_TILE
    issue_idx(ebase, 0)
    issue_idx(ebase + _B, 1)
    wait_idx(0)
    issue_gather(0)

    def cp_dstS(s):
        for g in range(_B // 16):
            dS[s][pl.ds(g * 16, 16)] = dst[s][pl.ds(g * 16, 16)]

    def edge_pair(p, carry):
        k0 = 2 * p
        wait_idx(1)

        @pl.when(p > 0)
        def _():
            wait_scatter(1)

        issue_gather(1)
        wait_gather(0)
        scale_rows(0)
        cp_dstS(0)
        issue_scatter(0)

        @pl.when(k0 + 2 < _EDGE_CHUNKS)
        def _():
            issue_idx(ebase + (k0 + 2) * _B, 0)

        wait_gather(1)
        scale_rows(1)
        cp_dstS(1)
        issue_scatter(1)

        @pl.when(k0 + 2 < _EDGE_CHUNKS)
        def _():
            wait_idx(0)
            wait_scatter(0)
            issue_gather(0)

        @pl.when(k0 + 3 < _EDGE_CHUNKS)
        def _():
            issue_idx(ebase + (k0 + 3) * _B, 1)

        return carry

    lax.fori_loop(0, _EDGE_CHUNKS // 2, edge_pair, 0)
    wait_scatter(0)
    wait_scatter(1)
    plsc.subcore_barrier()

    for j in range(8):
        ci = ss + j * _NS

        @pl.when(ci < _ROW_CHUNKS)
        def _():
            start = ci * _RB
            pltpu.sync_copy(tmp_sh.at[pl.ds(start, _RB)],
                            out_hbm.at[pl.ds(cc * _N + start, _RB)])


_R = 2000


def _tc_body(x_ref, t0_ref, t1_ref, w0_ref, w1_ref, bc_ref, wl_ref, bl_ref,
             out_ref):
    tx = t0_ref[...] + t1_ref[...]
    a = (jnp.dot(x_ref[...], w0_ref[...], preferred_element_type=jnp.float32)
         + jnp.dot(tx, w1_ref[...], preferred_element_type=jnp.float32)
         + bc_ref[...])
    z = jax.nn.sigmoid(a[:, :_F])
    ht = jnp.tanh(a[:, _F:])
    h = (1.0 - z) * ht
    out_ref[...] = (jnp.dot(jnp.maximum(h, 0.0), wl_ref[...],
                            preferred_element_type=jnp.float32) + bl_ref[...])


_tc_dense = pl.pallas_call(
    _tc_body,
    grid=(_N // _R,),
    in_specs=[
        pl.BlockSpec((_R, _F), lambda i: (i, 0)),
        pl.BlockSpec((_R, _F), lambda i: (i, 0)),
        pl.BlockSpec((_R, _F), lambda i: (i, 0)),
        pl.BlockSpec((_F, 2 * _F), lambda i: (0, 0)),
        pl.BlockSpec((_F, 2 * _F), lambda i: (0, 0)),
        pl.BlockSpec((1, 2 * _F), lambda i: (0, 0)),
        pl.BlockSpec((_F, 1), lambda i: (0, 0)),
        pl.BlockSpec((1, 1), lambda i: (0, 0)),
    ],
    out_specs=pl.BlockSpec((_R, 1), lambda i: (i, 0)),
    out_shape=jax.ShapeDtypeStruct((_N, 1), jnp.float32),
)


def kernel(x, edge_index, edge_weight, W0_xz, W1_xz, b_xz, W0_hz, W1_hz, b_hz,
           W0_xr, W1_xr, b_xr, W0_hr, W1_hr, b_hr, W0_xh, W1_xh, b_xh,
           W0_hh, W1_hh, b_hh, W_lin, b_lin):
    pad = _EP - _E
    padidx = (jnp.arange(pad, dtype=jnp.int32) % _N).astype(jnp.int32)
    src_p = jnp.concatenate([edge_index[0], padidx])
    dst_p = jnp.concatenate([edge_index[1], padidx])
    ew_p = jnp.concatenate([edge_weight, jnp.zeros((pad,), jnp.float32)])
    tmp = _sc_sparse(x, src_p, dst_p, ew_p)
    W0c = jnp.concatenate([W0_xz, W0_xh], axis=1)
    W1c = jnp.concatenate([W1_xz, W1_xh], axis=1)
    bc = jnp.concatenate([b_xz + b_hz, b_xh + b_hh]).reshape(1, 2 * _F)
    return _tc_dense(x, tmp[:_N], tmp[_N:], W0c, W1c, bc, W_lin,
                     b_lin.reshape(1, 1))


# speedup vs baseline: 41.1004x; 1.0071x over previous
import functools

import jax
import jax.numpy as jnp
from jax import lax
from jax.experimental import pallas as pl
from jax.experimental.pallas import tpu as pltpu
from jax.experimental.pallas import tpu_sc as plsc

_N = 10000
_E = 320000
_F = 128
_B = 128
_NC = 2
_NS = 16
_EP = 327680
_RB = 80
_ROW_CHUNKS = _N // _RB
_E_PER_TILE_DEG = _EP // _NS
_DEG_CHUNKS = _E_PER_TILE_DEG // _B
_E_PER_TILE = _EP // (_NC * _NS)
_EDGE_CHUNKS = _E_PER_TILE // _B


def _rsqrt16(d):
    i = lax.bitcast_convert_type(d, jnp.int32)
    i = jnp.int32(0x5F3759DF) - (i >> 1)
    r = lax.bitcast_convert_type(i, jnp.float32)
    for _ in range(4):
        r = r * (1.5 - 0.5 * d * r * r)
    return jnp.where(d > 0.0, r, 0.0)


_sc_mesh = plsc.VectorSubcoreMesh(core_axis_name="c", subcore_axis_name="s")


@functools.partial(
    pl.kernel,
    out_type=jax.ShapeDtypeStruct((_NC * _N, _F), jnp.float32),
    mesh=_sc_mesh,
    compiler_params=pltpu.CompilerParams(needs_layout_passes=False),
    scratch_types=(
        [pltpu.VMEM((_B,), jnp.int32) for _ in range(2)]
        + [pltpu.VMEM((_B,), jnp.int32) for _ in range(2)]
        + [pltpu.VMEM((_B,), jnp.float32) for _ in range(2)]
        + [pltpu.VMEM((_B,), jnp.int32) for _ in range(2)]
        + [pltpu.VMEM((_B,), jnp.float32) for _ in range(2)]
        + [pltpu.VMEM((_B, _F), jnp.float32) for _ in range(2)]
        + [
            pltpu.VMEM((_RB,), jnp.float32),
            pltpu.VMEM((_RB,), jnp.float32),
            pltpu.VMEM((_N,), jnp.float32),
            pltpu.VMEM_SHARED((_N, _F), jnp.float32),
            pltpu.VMEM_SHARED((_N,), jnp.float32),
            pltpu.VMEM_SHARED((_N,), jnp.float32),
        ]
        + [pltpu.SemaphoreType.DMA for _ in range(2)]
        + [pltpu.SemaphoreType.DMA for _ in range(2)]
        + [pltpu.SemaphoreType.DMA for _ in range(2)]
    ),
)
def _sc_sparse(x_hbm, src_hbm, dst_hbm, ew_hbm, out_hbm,
               src_0, src_1, dst_0, dst_1, ew_0, ew_1,
               dS_0, dS_1, eS_0, eS_1, rw_0, rw_1,
               z80a, z80b, dis_v, tmp_sh, deg_sh, dis_sh,
               sA_0, sA_1, sG_0, sG_1, sS_0, sS_1):
    src = [src_0, src_1]
    dst = [dst_0, dst_1]
    ew = [ew_0, ew_1]
    dS = [dS_0, dS_1]
    eS = [eS_0, eS_1]
    rows = [rw_0, rw_1]
    semA = [sA_0, sA_1]
    semG = [sG_0, sG_1]
    semS = [sS_0, sS_1]

    cc = lax.axis_index("c")
    ss = lax.axis_index("s")

    def issue_idx(base, s):
        pltpu.async_copy(src_hbm.at[pl.ds(base, _B)], src[s], semA[s])
        pltpu.async_copy(dst_hbm.at[pl.ds(base, _B)], dst[s], semA[s])
        pltpu.async_copy(ew_hbm.at[pl.ds(base, _B)], ew[s], semA[s])

    def wait_idx(s):
        pltpu.make_async_copy(src_hbm.at[pl.ds(0, _B)], src[s], semA[s]).wait()
        pltpu.make_async_copy(dst_hbm.at[pl.ds(0, _B)], dst[s], semA[s]).wait()
        pltpu.make_async_copy(ew_hbm.at[pl.ds(0, _B)], ew[s], semA[s]).wait()

    def issue_deg(base, s):
        pltpu.async_copy(src_hbm.at[pl.ds(base, _B)], src[s], semA[s])
        pltpu.async_copy(ew_hbm.at[pl.ds(base, _B)], ew[s], semA[s])

    def wait_deg(s):
        pltpu.make_async_copy(src_hbm.at[pl.ds(0, _B)], src[s], semA[s]).wait()
        pltpu.make_async_copy(ew_hbm.at[pl.ds(0, _B)], ew[s], semA[s]).wait()

    def issue_gather(s):
        pltpu.async_copy(x_hbm.at[src[s]], rows[s], semG[s])

    def wait_gather(s):
        pltpu.make_async_copy(x_hbm.at[src[s]], rows[s], semG[s]).wait()

    def issue_scatter(s):
        pltpu.async_copy(rows[s], tmp_sh.at[dS[s]], semS[s], add=True)

    def wait_scatter(s):
        pltpu.make_async_copy(rows[s], tmp_sh.at[dS[s]], semS[s]).wait()

    def scale_rows(s):
        def grp(g, carry):
            sv = src[s][pl.ds(g * 16, 16)]
            dv = dst[s][pl.ds(g * 16, 16)]
            lh = -(plsc.load_gather(dis_v, [sv])
                   * ew[s][pl.ds(g * 16, 16)]
                   * plsc.load_gather(dis_v, [dv]))
            for i in range(16):
                sc = lh[i]
                r = g * 16 + i
                for jj in range(_F // 16):
                    rows[s][r, pl.ds(jj * 16, 16)] = (
                        rows[s][r, pl.ds(jj * 16, 16)] * sc)
            return carry

        lax.fori_loop(0, _B // 16, grp, 0)

    for g in range(_RB // 16):
        z80a[pl.ds(g * 16, 16)] = jnp.zeros((16,), jnp.float32)

    def _zero_rows(r, carry):
        for j in range(_F // 16):
            rw_0[r, pl.ds(j * 16, 16)] = jnp.zeros((16,), jnp.float32)
        return carry

    lax.fori_loop(0, _RB, _zero_rows, 0)

    for j in range(8):
        ci = ss + j * _NS

        @pl.when(ci < _ROW_CHUNKS)
        def _():
            start = ci * _RB
            pltpu.sync_copy(rw_0.at[pl.ds(0, _RB)],
                            tmp_sh.at[pl.ds(start, _RB)])
            pltpu.sync_copy(z80a, deg_sh.at[pl.ds(start, _RB)])

    plsc.subcore_barrier()

    dbase = ss * _E_PER_TILE_DEG
    issue_deg(dbase, 0)
    issue_deg(dbase + _B, 1)

    def deg_wait_scat(s):
        pltpu.make_async_copy(eS[s], deg_sh.at[dS[s]], semS[s]).wait()

    def deg_chunk(k, s):
        wait_deg(s)
        for g in range(_B // 16):
            dS[s][pl.ds(g * 16, 16)] = src[s][pl.ds(g * 16, 16)]
            eS[s][pl.ds(g * 16, 16)] = ew[s][pl.ds(g * 16, 16)]

        @pl.when(k + 2 < _DEG_CHUNKS)
        def _():
            issue_deg(dbase + (k + 2) * _B, s)

    def deg_pair(p, carry):
        k0 = 2 * p
        deg_chunk(k0, 0)

        @pl.when(p > 0)
        def _():
            deg_wait_scat(1)

        pltpu.async_copy(eS[0], deg_sh.at[dS[0]], semS[0], add=True)
        deg_chunk(k0 + 1, 1)
        deg_wait_scat(0)
        pltpu.async_copy(eS[1], deg_sh.at[dS[1]], semS[1], add=True)
        return carry

    lax.fori_loop(0, _DEG_CHUNKS // 2, deg_pair, 0)
    deg_wait_scat(1)
    plsc.subcore_barrier()

    for j in range(8):
        ci = ss + j * _NS

        @pl.when(ci < _ROW_CHUNKS)
        def _():
            start = ci * _RB
            pltpu.sync_copy(deg_sh.at[pl.ds(start, _RB)], z80a)
            for g in range(_RB // 16):
                z80b[pl.ds(g * 16, 16)] = _rsqrt16(z80a[pl.ds(g * 16, 16)])
            pltpu.sync_copy(z80b, dis_sh.at[pl.ds(start, _RB)])

    plsc.subcore_barrier()

    pltpu.sync_copy(dis_sh, dis_v)

    ebase = cc * (_EP // _NC) + ss * _E_PER_TILE
    issue_idx(ebase, 0)
    issue_idx(ebase + _B, 1)
    wait_idx(0)
    issue_gather(0)

    def cp_dstS(s):
        for g in range(_B // 16):
            dS[s][pl.ds(g * 16, 16)] = dst[s][pl.ds(g * 16, 16)]

    def edge_pair(p, carry):
        k0 = 2 * p
        wait_idx(1)

        @pl.when(p > 0)
        def _():
            wait_scatter(1)

        issue_gather(1)
        wait_gather(0)
        scale_rows(0)
        cp_dstS(0)
        issue_scatter(0)

        @pl.when(k0 + 2 < _EDGE_CHUNKS)
        def _():
            issue_idx(ebase + (k0 + 2) * _B, 0)

        wait_gather(1)
        scale_rows(1)
        cp_dstS(1)
        wait_scatter(0)
        issue_scatter(1)

        @pl.when(k0 + 2 < _EDGE_CHUNKS)
        def _():
            wait_idx(0)
            issue_gather(0)

        @pl.when(k0 + 3 < _EDGE_CHUNKS)
        def _():
            issue_idx(ebase + (k0 + 3) * _B, 1)

        return carry

    lax.fori_loop(0, _EDGE_CHUNKS // 2, edge_pair, 0)
    wait_scatter(1)
    plsc.subcore_barrier()

    for j in range(8):
        ci = ss + j * _NS

        @pl.when(ci < _ROW_CHUNKS)
        def _():
            start = ci * _RB
            pltpu.sync_copy(tmp_sh.at[pl.ds(start, _RB)],
                            out_hbm.at[pl.ds(cc * _N + start, _RB)])


_R = 2000


def _tc_body(x_ref, t0_ref, t1_ref, w0_ref, w1_ref, bc_ref, wl_ref, bl_ref,
             out_ref):
    tx = t0_ref[...] + t1_ref[...]
    a = (jnp.dot(x_ref[...], w0_ref[...], preferred_element_type=jnp.float32)
         + jnp.dot(tx, w1_ref[...], preferred_element_type=jnp.float32)
         + bc_ref[...])
    z = jax.nn.sigmoid(a[:, :_F])
    ht = jnp.tanh(a[:, _F:])
    h = (1.0 - z) * ht
    out_ref[...] = (jnp.dot(jnp.maximum(h, 0.0), wl_ref[...],
                            preferred_element_type=jnp.float32) + bl_ref[...])


_tc_dense = pl.pallas_call(
    _tc_body,
    grid=(_N // _R,),
    in_specs=[
        pl.BlockSpec((_R, _F), lambda i: (i, 0)),
        pl.BlockSpec((_R, _F), lambda i: (i, 0)),
        pl.BlockSpec((_R, _F), lambda i: (i, 0)),
        pl.BlockSpec((_F, 2 * _F), lambda i: (0, 0)),
        pl.BlockSpec((_F, 2 * _F), lambda i: (0, 0)),
        pl.BlockSpec((1, 2 * _F), lambda i: (0, 0)),
        pl.BlockSpec((_F, 1), lambda i: (0, 0)),
        pl.BlockSpec((1, 1), lambda i: (0, 0)),
    ],
    out_specs=pl.BlockSpec((_R, 1), lambda i: (i, 0)),
    out_shape=jax.ShapeDtypeStruct((_N, 1), jnp.float32),
)


def kernel(x, edge_index, edge_weight, W0_xz, W1_xz, b_xz, W0_hz, W1_hz, b_hz,
           W0_xr, W1_xr, b_xr, W0_hr, W1_hr, b_hr, W0_xh, W1_xh, b_xh,
           W0_hh, W1_hh, b_hh, W_lin, b_lin):
    pad = _EP - _E
    padidx = (jnp.arange(pad, dtype=jnp.int32) % _N).astype(jnp.int32)
    src_p = jnp.concatenate([edge_index[0], padidx])
    dst_p = jnp.concatenate([edge_index[1], padidx])
    ew_p = jnp.concatenate([edge_weight, jnp.zeros((pad,), jnp.float32)])
    tmp = _sc_sparse(x, src_p, dst_p, ew_p)
    W0c = jnp.concatenate([W0_xz, W0_xh], axis=1)
    W1c = jnp.concatenate([W1_xz, W1_xh], axis=1)
    bc = jnp.concatenate([b_xz + b_hz, b_xh + b_hh]).reshape(1, 2 * _F)
    return _tc_dense(x, tmp[:_N], tmp[_N:], W0c, W1c, bc, W_lin,
                     b_lin.reshape(1, 1))


# speedup vs baseline: 41.6378x; 1.0131x over previous
import functools

import jax
import jax.numpy as jnp
from jax import lax
from jax.experimental import pallas as pl
from jax.experimental.pallas import tpu as pltpu
from jax.experimental.pallas import tpu_sc as plsc

_N = 10000
_E = 320000
_F = 128
_B = 128
_NC = 2
_NS = 16
_EP = 327680
_RB = 80
_ROW_CHUNKS = _N // _RB
_E_PER_TILE = _EP // (_NC * _NS)
_CHUNKS = _E_PER_TILE // _B

_sc_mesh = plsc.VectorSubcoreMesh(core_axis_name="c", subcore_axis_name="s")


@functools.partial(
    pl.kernel,
    out_type=jax.ShapeDtypeStruct((_NC * _N,), jnp.float32),
    mesh=_sc_mesh,
    compiler_params=pltpu.CompilerParams(needs_layout_passes=False),
    scratch_types=(
        [pltpu.VMEM((_B,), jnp.int32) for _ in range(2)]
        + [pltpu.VMEM((_B,), jnp.float32) for _ in range(2)]
        + [pltpu.VMEM((_B,), jnp.int32) for _ in range(2)]
        + [pltpu.VMEM((_B,), jnp.float32) for _ in range(2)]
        + [
            pltpu.VMEM((_RB,), jnp.float32),
            pltpu.VMEM_SHARED((_N,), jnp.float32),
        ]
        + [pltpu.SemaphoreType.DMA for _ in range(2)]
        + [pltpu.SemaphoreType.DMA for _ in range(2)]
    ),
)
def _sc_deg(src_hbm, ew_hbm, out_hbm,
            src_0, src_1, ew_0, ew_1, dS_0, dS_1, eS_0, eS_1,
            z80, deg_sh, sA_0, sA_1, sS_0, sS_1):
    src = [src_0, src_1]
    ew = [ew_0, ew_1]
    dS = [dS_0, dS_1]
    eS = [eS_0, eS_1]
    semA = [sA_0, sA_1]
    semS = [sS_0, sS_1]

    cc = lax.axis_index("c")
    ss = lax.axis_index("s")

    def issue_deg(base, s):
        pltpu.async_copy(src_hbm.at[pl.ds(base, _B)], src[s], semA[s])
        pltpu.async_copy(ew_hbm.at[pl.ds(base, _B)], ew[s], semA[s])

    def wait_deg(s):
        pltpu.make_async_copy(src_hbm.at[pl.ds(0, _B)], src[s], semA[s]).wait()
        pltpu.make_async_copy(ew_hbm.at[pl.ds(0, _B)], ew[s], semA[s]).wait()

    def deg_wait_scat(s):
        pltpu.make_async_copy(eS[s], deg_sh.at[dS[s]], semS[s]).wait()

    for g in range(_RB // 16):
        z80[pl.ds(g * 16, 16)] = jnp.zeros((16,), jnp.float32)
    for j in range(8):
        ci = ss + j * _NS

        @pl.when(ci < _ROW_CHUNKS)
        def _():
            pltpu.sync_copy(z80, deg_sh.at[pl.ds(ci * _RB, _RB)])

    plsc.subcore_barrier()

    dbase = cc * (_EP // _NC) + ss * _E_PER_TILE
    issue_deg(dbase, 0)
    issue_deg(dbase + _B, 1)

    def deg_chunk(k, s):
        wait_deg(s)
        for g in range(_B // 16):
            dS[s][pl.ds(g * 16, 16)] = src[s][pl.ds(g * 16, 16)]
            eS[s][pl.ds(g * 16, 16)] = ew[s][pl.ds(g * 16, 16)]

        @pl.when(k + 2 < _CHUNKS)
        def _():
            issue_deg(dbase + (k + 2) * _B, s)

    def deg_pair(p, carry):
        k0 = 2 * p
        deg_chunk(k0, 0)

        @pl.when(p > 0)
        def _():
            deg_wait_scat(1)

        pltpu.async_copy(eS[0], deg_sh.at[dS[0]], semS[0], add=True)
        deg_chunk(k0 + 1, 1)
        deg_wait_scat(0)
        pltpu.async_copy(eS[1], deg_sh.at[dS[1]], semS[1], add=True)
        return carry

    lax.fori_loop(0, _CHUNKS // 2, deg_pair, 0)
    deg_wait_scat(1)
    plsc.subcore_barrier()

    for j in range(8):
        ci = ss + j * _NS

        @pl.when(ci < _ROW_CHUNKS)
        def _():
            start = ci * _RB
            pltpu.sync_copy(deg_sh.at[pl.ds(start, _RB)], z80)
            pltpu.sync_copy(z80, out_hbm.at[pl.ds(cc * _N + start, _RB)])


@functools.partial(
    pl.kernel,
    out_type=jax.ShapeDtypeStruct((_NC * _N, _F), jnp.float32),
    mesh=_sc_mesh,
    compiler_params=pltpu.CompilerParams(needs_layout_passes=False),
    scratch_types=(
        [pltpu.VMEM((_B,), jnp.int32) for _ in range(2)]
        + [pltpu.VMEM((_B,), jnp.int32) for _ in range(2)]
        + [pltpu.VMEM((_B,), jnp.float32) for _ in range(2)]
        + [pltpu.VMEM((_B,), jnp.int32) for _ in range(2)]
        + [pltpu.VMEM((_B, _F), jnp.float32) for _ in range(2)]
        + [
            pltpu.VMEM_SHARED((_N, _F), jnp.float32),
        ]
        + [pltpu.SemaphoreType.DMA for _ in range(2)]
        + [pltpu.SemaphoreType.DMA for _ in range(2)]
        + [pltpu.SemaphoreType.DMA for _ in range(2)]
        + [pltpu.SemaphoreType.DMA]
    ),
)
def _sc_edges(y_hbm, src_hbm, dst_hbm, ew_hbm, out_hbm,
              src_0, src_1, dst_0, dst_1, ew_0, ew_1, dS_0, dS_1,
              rw_0, rw_1, tmp_sh,
              sA_0, sA_1, sG_0, sG_1, sS_0, sS_1, semZ):
    src = [src_0, src_1]
    dst = [dst_0, dst_1]
    ew = [ew_0, ew_1]
    dS = [dS_0, dS_1]
    rows = [rw_0, rw_1]
    semA = [sA_0, sA_1]
    semG = [sG_0, sG_1]
    semS = [sS_0, sS_1]

    cc = lax.axis_index("c")
    ss = lax.axis_index("s")

    def issue_idx(base, s):
        pltpu.async_copy(src_hbm.at[pl.ds(base, _B)], src[s], semA[s])
        pltpu.async_copy(dst_hbm.at[pl.ds(base, _B)], dst[s], semA[s])
        pltpu.async_copy(ew_hbm.at[pl.ds(base, _B)], ew[s], semA[s])

    def wait_idx(s):
        pltpu.make_async_copy(src_hbm.at[pl.ds(0, _B)], src[s], semA[s]).wait()
        pltpu.make_async_copy(dst_hbm.at[pl.ds(0, _B)], dst[s], semA[s]).wait()
        pltpu.make_async_copy(ew_hbm.at[pl.ds(0, _B)], ew[s], semA[s]).wait()

    def issue_gather(s):
        pltpu.async_copy(y_hbm.at[src[s]], rows[s], semG[s])

    def wait_gather(s):
        pltpu.make_async_copy(y_hbm.at[src[s]], rows[s], semG[s]).wait()

    def issue_scatter(s):
        pltpu.async_copy(rows[s], tmp_sh.at[dS[s]], semS[s], add=True)

    def wait_scatter(s):
        pltpu.make_async_copy(rows[s], tmp_sh.at[dS[s]], semS[s]).wait()

    def scale_rows(s):
        def grp(g, carry):
            lh = ew[s][pl.ds(g * 16, 16)]
            for i in range(16):
                sc = lh[i]
                r = g * 16 + i
                for jj in range(_F // 16):
                    rows[s][r, pl.ds(jj * 16, 16)] = (
                        rows[s][r, pl.ds(jj * 16, 16)] * sc)
            return carry

        lax.fori_loop(0, _B // 16, grp, 0)

    def cp_dstS(s):
        for g in range(_B // 16):
            dS[s][pl.ds(g * 16, 16)] = dst[s][pl.ds(g * 16, 16)]

    def _zero_rows(r, carry):
        for j in range(_F // 16):
            rw_0[r, pl.ds(j * 16, 16)] = jnp.zeros((16,), jnp.float32)
        return carry

    lax.fori_loop(0, _RB, _zero_rows, 0)

    for j in range(8):
        ci = ss + j * _NS

        @pl.when(ci < _ROW_CHUNKS)
        def _():
            pltpu.async_copy(rw_0.at[pl.ds(0, _RB)],
                             tmp_sh.at[pl.ds(ci * _RB, _RB)], semZ)

    ebase = cc * (_EP // _NC) + ss * _E_PER_TILE
    issue_idx(ebase, 0)
    issue_idx(ebase + _B, 1)
    wait_idx(0)
    issue_gather(0)

    for j in range(8):
        ci = ss + j * _NS

        @pl.when(ci < _ROW_CHUNKS)
        def _():
            pltpu.make_async_copy(rw_0.at[pl.ds(0, _RB)],
                                  tmp_sh.at[pl.ds(0, _RB)], semZ).wait()

    plsc.subcore_barrier()

    def edge_pair(p, carry):
        k0 = 2 * p
        wait_idx(1)

        @pl.when(p > 0)
        def _():
            wait_scatter(1)

        issue_gather(1)
        wait_gather(0)
        scale_rows(0)
        cp_dstS(0)
        issue_scatter(0)

        @pl.when(k0 + 2 < _CHUNKS)
        def _():
            issue_idx(ebase + (k0 + 2) * _B, 0)

        wait_gather(1)
        scale_rows(1)
        cp_dstS(1)
        wait_scatter(0)
        issue_scatter(1)

        @pl.when(k0 + 2 < _CHUNKS)
        def _():
            wait_idx(0)
            issue_gather(0)

        @pl.when(k0 + 3 < _CHUNKS)
        def _():
            issue_idx(ebase + (k0 + 3) * _B, 1)

        return carry

    lax.fori_loop(0, _CHUNKS // 2, edge_pair, 0)
    wait_scatter(1)
    plsc.subcore_barrier()

    for j in range(8):
        ci = ss + j * _NS

        @pl.when(ci < _ROW_CHUNKS)
        def _():
            start = ci * _RB
            pltpu.sync_copy(tmp_sh.at[pl.ds(start, _RB)],
                            out_hbm.at[pl.ds(cc * _N + start, _RB)])


_R = 2000


def _tc_mid_body(d0_ref, d1_ref, x_ref, y_ref, dis_ref):
    d = d0_ref[...] + d1_ref[...]
    di = jnp.where(d > 0.0, lax.rsqrt(jnp.where(d > 0.0, d, 1.0)), 0.0)
    dis_ref[...] = di
    y_ref[...] = x_ref[...] * di


_tc_mid = pl.pallas_call(
    _tc_mid_body,
    grid=(_N // _R,),
    in_specs=[
        pl.BlockSpec((_R, 1), lambda i: (i, 0)),
        pl.BlockSpec((_R, 1), lambda i: (i + _N // _R, 0)),
        pl.BlockSpec((_R, _F), lambda i: (i, 0)),
    ],
    out_specs=[
        pl.BlockSpec((_R, _F), lambda i: (i, 0)),
        pl.BlockSpec((_R, 1), lambda i: (i, 0)),
    ],
    out_shape=[
        jax.ShapeDtypeStruct((_N, _F), jnp.float32),
        jax.ShapeDtypeStruct((_N, 1), jnp.float32),
    ],
)


def _tc_body(dis_ref, x_ref, t0_ref, t1_ref, w0_ref, w1_ref, bc_ref, wl_ref,
             bl_ref, out_ref):
    tx = (t0_ref[...] + t1_ref[...]) * (-dis_ref[...])
    a = (jnp.dot(x_ref[...], w0_ref[...], preferred_element_type=jnp.float32)
         + jnp.dot(tx, w1_ref[...], preferred_element_type=jnp.float32)
         + bc_ref[...])
    z = jax.nn.sigmoid(a[:, :_F])
    ht = jnp.tanh(a[:, _F:])
    h = (1.0 - z) * ht
    out_ref[...] = (jnp.dot(jnp.maximum(h, 0.0), wl_ref[...],
                            preferred_element_type=jnp.float32) + bl_ref[...])


_tc_dense = pl.pallas_call(
    _tc_body,
    grid=(_N // _R,),
    in_specs=[
        pl.BlockSpec((_R, 1), lambda i: (i, 0)),
        pl.BlockSpec((_R, _F), lambda i: (i, 0)),
        pl.BlockSpec((_R, _F), lambda i: (i, 0)),
        pl.BlockSpec((_R, _F), lambda i: (i, 0)),
        pl.BlockSpec((_F, 2 * _F), lambda i: (0, 0)),
        pl.BlockSpec((_F, 2 * _F), lambda i: (0, 0)),
        pl.BlockSpec((1, 2 * _F), lambda i: (0, 0)),
        pl.BlockSpec((_F, 1), lambda i: (0, 0)),
        pl.BlockSpec((1, 1), lambda i: (0, 0)),
    ],
    out_specs=pl.BlockSpec((_R, 1), lambda i: (i, 0)),
    out_shape=jax.ShapeDtypeStruct((_N, 1), jnp.float32),
)


def kernel(x, edge_index, edge_weight, W0_xz, W1_xz, b_xz, W0_hz, W1_hz, b_hz,
           W0_xr, W1_xr, b_xr, W0_hr, W1_hr, b_hr, W0_xh, W1_xh, b_xh,
           W0_hh, W1_hh, b_hh, W_lin, b_lin):
    pad = _EP - _E
    padidx = (jnp.arange(pad, dtype=jnp.int32) % _N).astype(jnp.int32)
    src_p = jnp.concatenate([edge_index[0], padidx])
    dst_p = jnp.concatenate([edge_index[1], padidx])
    ew_p = jnp.concatenate([edge_weight, jnp.zeros((pad,), jnp.float32)])
    deg = _sc_deg(src_p, ew_p).reshape(_NC * _N, 1)
    y, dis = _tc_mid(deg, deg, x)
    tmp = _sc_edges(y, src_p, dst_p, ew_p)
    W0c = jnp.concatenate([W0_xz, W0_xh], axis=1)
    W1c = jnp.concatenate([W1_xz, W1_xh], axis=1)
    bc = jnp.concatenate([b_xz + b_hz, b_xh + b_hh]).reshape(1, 2 * _F)
    return _tc_dense(dis, x, tmp[:_N], tmp[_N:], W0c, W1c, bc, W_lin,
                     b_lin.reshape(1, 1))


# speedup vs baseline: 41.6986x; 1.0015x over previous
import functools

import jax
import jax.numpy as jnp
from jax import lax
from jax.experimental import pallas as pl
from jax.experimental.pallas import tpu as pltpu
from jax.experimental.pallas import tpu_sc as plsc

_N = 10000
_E = 320000
_F = 128
_B = 128
_NC = 2
_NS = 16
_EP = 327680
_RB = 80
_ROW_CHUNKS = _N // _RB
_E_PER_TILE = _EP // (_NC * _NS)
_CHUNKS = _E_PER_TILE // _B

_sc_mesh = plsc.VectorSubcoreMesh(core_axis_name="c", subcore_axis_name="s")


@functools.partial(
    pl.kernel,
    out_type=jax.ShapeDtypeStruct((_NC * _N,), jnp.float32),
    mesh=_sc_mesh,
    compiler_params=pltpu.CompilerParams(needs_layout_passes=False),
    scratch_types=(
        [pltpu.VMEM((_B,), jnp.int32) for _ in range(2)]
        + [pltpu.VMEM((_B,), jnp.float32) for _ in range(2)]
        + [pltpu.VMEM((_B,), jnp.int32) for _ in range(2)]
        + [pltpu.VMEM((_B,), jnp.float32) for _ in range(2)]
        + [
            pltpu.VMEM((_RB,), jnp.float32),
            pltpu.VMEM_SHARED((_N,), jnp.float32),
        ]
        + [pltpu.SemaphoreType.DMA for _ in range(2)]
        + [pltpu.SemaphoreType.DMA for _ in range(2)]
    ),
)
def _sc_deg(src_hbm, ew_hbm, out_hbm,
            src_0, src_1, ew_0, ew_1, dS_0, dS_1, eS_0, eS_1,
            z80, deg_sh, sA_0, sA_1, sS_0, sS_1):
    src = [src_0, src_1]
    ew = [ew_0, ew_1]
    dS = [dS_0, dS_1]
    eS = [eS_0, eS_1]
    semA = [sA_0, sA_1]
    semS = [sS_0, sS_1]

    cc = lax.axis_index("c")
    ss = lax.axis_index("s")

    def issue_deg(base, s):
        pltpu.async_copy(src_hbm.at[pl.ds(base, _B)], src[s], semA[s])
        pltpu.async_copy(ew_hbm.at[pl.ds(base, _B)], ew[s], semA[s])

    def wait_deg(s):
        pltpu.make_async_copy(src_hbm.at[pl.ds(0, _B)], src[s], semA[s]).wait()
        pltpu.make_async_copy(ew_hbm.at[pl.ds(0, _B)], ew[s], semA[s]).wait()

    def deg_wait_scat(s):
        pltpu.make_async_copy(eS[s], deg_sh.at[dS[s]], semS[s]).wait()

    for g in range(_RB // 16):
        z80[pl.ds(g * 16, 16)] = jnp.zeros((16,), jnp.float32)
    for j in range(8):
        ci = ss + j * _NS

        @pl.when(ci < _ROW_CHUNKS)
        def _():
            pltpu.sync_copy(z80, deg_sh.at[pl.ds(ci * _RB, _RB)])

    plsc.subcore_barrier()

    dbase = cc * (_EP // _NC) + ss * _E_PER_TILE
    issue_deg(dbase, 0)
    issue_deg(dbase + _B, 1)

    def deg_chunk(k, s):
        wait_deg(s)
        for g in range(_B // 16):
            dS[s][pl.ds(g * 16, 16)] = src[s][pl.ds(g * 16, 16)]
            eS[s][pl.ds(g * 16, 16)] = ew[s][pl.ds(g * 16, 16)]

        @pl.when(k + 2 < _CHUNKS)
        def _():
            issue_deg(dbase + (k + 2) * _B, s)

    def deg_pair(p, carry):
        k0 = 2 * p
        deg_chunk(k0, 0)

        @pl.when(p > 0)
        def _():
            deg_wait_scat(1)

        pltpu.async_copy(eS[0], deg_sh.at[dS[0]], semS[0], add=True)
        deg_chunk(k0 + 1, 1)
        deg_wait_scat(0)
        pltpu.async_copy(eS[1], deg_sh.at[dS[1]], semS[1], add=True)
        return carry

    lax.fori_loop(0, _CHUNKS // 2, deg_pair, 0)
    deg_wait_scat(1)
    plsc.subcore_barrier()

    for j in range(8):
        ci = ss + j * _NS

        @pl.when(ci < _ROW_CHUNKS)
        def _():
            start = ci * _RB
            pltpu.sync_copy(deg_sh.at[pl.ds(start, _RB)], z80)
            pltpu.sync_copy(z80, out_hbm.at[pl.ds(cc * _N + start, _RB)])


@functools.partial(
    pl.kernel,
    out_type=jax.ShapeDtypeStruct((_NC * _N, _F), jnp.float32),
    mesh=_sc_mesh,
    compiler_params=pltpu.CompilerParams(needs_layout_passes=False),
    scratch_types=(
        [pltpu.VMEM((_B,), jnp.int32) for _ in range(2)]
        + [pltpu.VMEM((_B,), jnp.int32) for _ in range(2)]
        + [pltpu.VMEM((_B,), jnp.float32) for _ in range(2)]
        + [pltpu.VMEM((_B,), jnp.int32) for _ in range(2)]
        + [pltpu.VMEM((_B, _F), jnp.float32) for _ in range(2)]
        + [
            pltpu.VMEM_SHARED((_N, _F), jnp.float32),
        ]
        + [pltpu.SemaphoreType.DMA for _ in range(2)]
        + [pltpu.SemaphoreType.DMA for _ in range(2)]
        + [pltpu.SemaphoreType.DMA for _ in range(2)]
        + [pltpu.SemaphoreType.DMA]
    ),
)
def _sc_edges(y_hbm, src_hbm, dst_hbm, ew_hbm, out_hbm,
              src_0, src_1, dst_0, dst_1, ew_0, ew_1, dS_0, dS_1,
              rw_0, rw_1, tmp_sh,
              sA_0, sA_1, sG_0, sG_1, sS_0, sS_1, semZ):
    src = [src_0, src_1]
    dst = [dst_0, dst_1]
    ew = [ew_0, ew_1]
    dS = [dS_0, dS_1]
    rows = [rw_0, rw_1]
    semA = [sA_0, sA_1]
    semG = [sG_0, sG_1]
    semS = [sS_0, sS_1]

    cc = lax.axis_index("c")
    ss = lax.axis_index("s")

    def issue_idx(base, s):
        pltpu.async_copy(src_hbm.at[pl.ds(base, _B)], src[s], semA[s])
        pltpu.async_copy(dst_hbm.at[pl.ds(base, _B)], dst[s], semA[s])
        pltpu.async_copy(ew_hbm.at[pl.ds(base, _B)], ew[s], semA[s])

    def wait_idx(s):
        pltpu.make_async_copy(src_hbm.at[pl.ds(0, _B)], src[s], semA[s]).wait()
        pltpu.make_async_copy(dst_hbm.at[pl.ds(0, _B)], dst[s], semA[s]).wait()
        pltpu.make_async_copy(ew_hbm.at[pl.ds(0, _B)], ew[s], semA[s]).wait()

    def issue_gather(s):
        pltpu.async_copy(y_hbm.at[src[s]], rows[s], semG[s])

    def wait_gather(s):
        pltpu.make_async_copy(y_hbm.at[src[s]], rows[s], semG[s]).wait()

    def issue_scatter(s):
        pltpu.async_copy(rows[s], tmp_sh.at[dS[s]], semS[s], add=True)

    def wait_scatter(s):
        pltpu.make_async_copy(rows[s], tmp_sh.at[dS[s]], semS[s]).wait()

    def scale_rows(s):
        def grp(g, carry):
            lh = ew[s][pl.ds(g * 16, 16)]
            for i in range(16):
                sc = lh[i]
                r = g * 16 + i
                for jj in range(_F // 16):
                    rows[s][r, pl.ds(jj * 16, 16)] = (
                        rows[s][r, pl.ds(jj * 16, 16)] * sc)
            return carry

        lax.fori_loop(0, _B // 16, grp, 0)

    def cp_dstS(s):
        for g in range(_B // 16):
            dS[s][pl.ds(g * 16, 16)] = dst[s][pl.ds(g * 16, 16)]

    def _zero_rows(r, carry):
        for j in range(_F // 16):
            rw_1[r, pl.ds(j * 16, 16)] = jnp.zeros((16,), jnp.float32)
        return carry

    lax.fori_loop(0, _RB, _zero_rows, 0)

    for j in range(8):
        ci = ss + j * _NS

        @pl.when(ci < _ROW_CHUNKS)
        def _():
            pltpu.async_copy(rw_1.at[pl.ds(0, _RB)],
                             tmp_sh.at[pl.ds(ci * _RB, _RB)], semZ)

    ebase = cc * (_EP // _NC) + ss * _E_PER_TILE
    issue_idx(ebase, 0)
    issue_idx(ebase + _B, 1)
    wait_idx(0)
    issue_gather(0)

    for j in range(8):
        ci = ss + j * _NS

        @pl.when(ci < _ROW_CHUNKS)
        def _():
            pltpu.make_async_copy(rw_1.at[pl.ds(0, _RB)],
                                  tmp_sh.at[pl.ds(0, _RB)], semZ).wait()

    plsc.subcore_barrier()

    def edge_pair(p, carry):
        k0 = 2 * p
        wait_idx(1)

        @pl.when(p > 0)
        def _():
            wait_scatter(1)

        issue_gather(1)
        wait_gather(0)
        scale_rows(0)
        cp_dstS(0)
        issue_scatter(0)

        @pl.when(k0 + 2 < _CHUNKS)
        def _():
            issue_idx(ebase + (k0 + 2) * _B, 0)

        wait_gather(1)
        scale_rows(1)
        cp_dstS(1)
        wait_scatter(0)
        issue_scatter(1)

        @pl.when(k0 + 2 < _CHUNKS)
        def _():
            wait_idx(0)
            issue_gather(0)

        @pl.when(k0 + 3 < _CHUNKS)
        def _():
            issue_idx(ebase + (k0 + 3) * _B, 1)

        return carry

    lax.fori_loop(0, _CHUNKS // 2, edge_pair, 0)
    wait_scatter(1)
    plsc.subcore_barrier()

    for j in range(8):
        ci = ss + j * _NS

        @pl.when(ci < _ROW_CHUNKS)
        def _():
            start = ci * _RB
            pltpu.sync_copy(tmp_sh.at[pl.ds(start, _RB)],
                            out_hbm.at[pl.ds(cc * _N + start, _RB)])


_R = 2000


def _tc_mid_body(d0_ref, d1_ref, x_ref, y_ref, dis_ref):
    d = d0_ref[...] + d1_ref[...]
    di = jnp.where(d > 0.0, lax.rsqrt(jnp.where(d > 0.0, d, 1.0)), 0.0)
    dis_ref[...] = di
    y_ref[...] = x_ref[...] * di


_tc_mid = pl.pallas_call(
    _tc_mid_body,
    grid=(_N // _R,),
    in_specs=[
        pl.BlockSpec((_R, 1), lambda i: (i, 0)),
        pl.BlockSpec((_R, 1), lambda i: (i + _N // _R, 0)),
        pl.BlockSpec((_R, _F), lambda i: (i, 0)),
    ],
    out_specs=[
        pl.BlockSpec((_R, _F), lambda i: (i, 0)),
        pl.BlockSpec((_R, 1), lambda i: (i, 0)),
    ],
    out_shape=[
        jax.ShapeDtypeStruct((_N, _F), jnp.float32),
        jax.ShapeDtypeStruct((_N, 1), jnp.float32),
    ],
)


def _tc_body(dis_ref, x_ref, t0_ref, t1_ref, w0_ref, w1_ref, bc_ref, wl_ref,
             bl_ref, out_ref):
    tx = (t0_ref[...] + t1_ref[...]) * (-dis_ref[...])
    a = (jnp.dot(x_ref[...], w0_ref[...], preferred_element_type=jnp.float32)
         + jnp.dot(tx, w1_ref[...], preferred_element_type=jnp.float32)
         + bc_ref[...])
    z = jax.nn.sigmoid(a[:, :_F])
    ht = jnp.tanh(a[:, _F:])
    h = (1.0 - z) * ht
    out_ref[...] = (jnp.dot(jnp.maximum(h, 0.0), wl_ref[...],
                            preferred_element_type=jnp.float32) + bl_ref[...])


_tc_dense = pl.pallas_call(
    _tc_body,
    grid=(_N // _R,),
    in_specs=[
        pl.BlockSpec((_R, 1), lambda i: (i, 0)),
        pl.BlockSpec((_R, _F), lambda i: (i, 0)),
        pl.BlockSpec((_R, _F), lambda i: (i, 0)),
        pl.BlockSpec((_R, _F), lambda i: (i, 0)),
        pl.BlockSpec((_F, 2 * _F), lambda i: (0, 0)),
        pl.BlockSpec((_F, 2 * _F), lambda i: (0, 0)),
        pl.BlockSpec((1, 2 * _F), lambda i: (0, 0)),
        pl.BlockSpec((_F, 1), lambda i: (0, 0)),
        pl.BlockSpec((1, 1), lambda i: (0, 0)),
    ],
    out_specs=pl.BlockSpec((_R, 1), lambda i: (i, 0)),
    out_shape=jax.ShapeDtypeStruct((_N, 1), jnp.float32),
)


def kernel(x, edge_index, edge_weight, W0_xz, W1_xz, b_xz, W0_hz, W1_hz, b_hz,
           W0_xr, W1_xr, b_xr, W0_hr, W1_hr, b_hr, W0_xh, W1_xh, b_xh,
           W0_hh, W1_hh, b_hh, W_lin, b_lin):
    pad = _EP - _E
    padidx = (jnp.arange(pad, dtype=jnp.int32) % _N).astype(jnp.int32)
    src_p = jnp.concatenate([edge_index[0], padidx])
    dst_p = jnp.concatenate([edge_index[1], padidx])
    ew_p = jnp.concatenate([edge_weight, jnp.zeros((pad,), jnp.float32)])
    deg = _sc_deg(src_p, ew_p).reshape(_NC * _N, 1)
    y, dis = _tc_mid(deg, deg, x)
    tmp = _sc_edges(y, src_p, dst_p, ew_p)
    W0c = jnp.concatenate([W0_xz, W0_xh], axis=1)
    W1c = jnp.concatenate([W1_xz, W1_xh], axis=1)
    bc = jnp.concatenate([b_xz + b_hz, b_xh + b_hh]).reshape(1, 2 * _F)
    return _tc_dense(dis, x, tmp[:_N], tmp[_N:], W0c, W1c, bc, W_lin,
                     b_lin.reshape(1, 1))


# speedup vs baseline: 45.5683x; 1.0928x over previous
import functools

import jax
import jax.numpy as jnp
from jax import lax
from jax.experimental import pallas as pl
from jax.experimental.pallas import tpu as pltpu
from jax.experimental.pallas import tpu_sc as plsc

_N = 10000
_E = 320000
_F = 128
_B = 128
_NC = 2
_NS = 16
_EP = 327680
_RB = 80
_ROW_CHUNKS = _N // _RB
_E_PER_TILE = _EP // (_NC * _NS)
_CHUNKS = _E_PER_TILE // _B

_sc_mesh = plsc.VectorSubcoreMesh(core_axis_name="c", subcore_axis_name="s")


@functools.partial(
    pl.kernel,
    out_type=jax.ShapeDtypeStruct((_NC * _N,), jnp.float32),
    mesh=_sc_mesh,
    compiler_params=pltpu.CompilerParams(needs_layout_passes=False),
    scratch_types=(
        [pltpu.VMEM((_B,), jnp.int32) for _ in range(2)]
        + [pltpu.VMEM((_B,), jnp.float32) for _ in range(2)]
        + [pltpu.VMEM((_B,), jnp.int32) for _ in range(2)]
        + [pltpu.VMEM((_B,), jnp.float32) for _ in range(2)]
        + [
            pltpu.VMEM((_RB,), jnp.float32),
            pltpu.VMEM_SHARED((_N,), jnp.float32),
        ]
        + [pltpu.SemaphoreType.DMA for _ in range(2)]
        + [pltpu.SemaphoreType.DMA for _ in range(2)]
    ),
)
def _sc_deg(src_hbm, ew_hbm, out_hbm,
            src_0, src_1, ew_0, ew_1, dS_0, dS_1, eS_0, eS_1,
            z80, deg_sh, sA_0, sA_1, sS_0, sS_1):
    src = [src_0, src_1]
    ew = [ew_0, ew_1]
    dS = [dS_0, dS_1]
    eS = [eS_0, eS_1]
    semA = [sA_0, sA_1]
    semS = [sS_0, sS_1]

    cc = lax.axis_index("c")
    ss = lax.axis_index("s")

    def issue_deg(base, s):
        pltpu.async_copy(src_hbm.at[pl.ds(base, _B)], src[s], semA[s])
        pltpu.async_copy(ew_hbm.at[pl.ds(base, _B)], ew[s], semA[s])

    def wait_deg(s):
        pltpu.make_async_copy(src_hbm.at[pl.ds(0, _B)], src[s], semA[s]).wait()
        pltpu.make_async_copy(ew_hbm.at[pl.ds(0, _B)], ew[s], semA[s]).wait()

    def deg_wait_scat(s):
        pltpu.make_async_copy(eS[s], deg_sh.at[dS[s]], semS[s]).wait()

    for g in range(_RB // 16):
        z80[pl.ds(g * 16, 16)] = jnp.zeros((16,), jnp.float32)
    for j in range(8):
        ci = ss + j * _NS

        @pl.when(ci < _ROW_CHUNKS)
        def _():
            pltpu.sync_copy(z80, deg_sh.at[pl.ds(ci * _RB, _RB)])

    plsc.subcore_barrier()

    dbase = cc * (_EP // _NC) + ss * _E_PER_TILE
    issue_deg(dbase, 0)
    issue_deg(dbase + _B, 1)

    def deg_chunk(k, s):
        wait_deg(s)
        for g in range(_B // 16):
            dS[s][pl.ds(g * 16, 16)] = src[s][pl.ds(g * 16, 16)]
            eS[s][pl.ds(g * 16, 16)] = ew[s][pl.ds(g * 16, 16)]

        @pl.when(k + 2 < _CHUNKS)
        def _():
            issue_deg(dbase + (k + 2) * _B, s)

    def deg_pair(p, carry):
        k0 = 2 * p
        deg_chunk(k0, 0)

        @pl.when(p > 0)
        def _():
            deg_wait_scat(1)

        pltpu.async_copy(eS[0], deg_sh.at[dS[0]], semS[0], add=True)
        deg_chunk(k0 + 1, 1)
        deg_wait_scat(0)
        pltpu.async_copy(eS[1], deg_sh.at[dS[1]], semS[1], add=True)
        return carry

    lax.fori_loop(0, _CHUNKS // 2, deg_pair, 0)
    deg_wait_scat(1)
    plsc.subcore_barrier()

    for j in range(8):
        ci = ss + j * _NS

        @pl.when(ci < _ROW_CHUNKS)
        def _():
            start = ci * _RB
            pltpu.sync_copy(deg_sh.at[pl.ds(start, _RB)], z80)
            pltpu.sync_copy(z80, out_hbm.at[pl.ds(cc * _N + start, _RB)])


@functools.partial(
    pl.kernel,
    out_type=jax.ShapeDtypeStruct((_NC * _N, _F), jnp.float32),
    mesh=_sc_mesh,
    compiler_params=pltpu.CompilerParams(needs_layout_passes=False),
    scratch_types=(
        [pltpu.VMEM((_B,), jnp.int32) for _ in range(3)]
        + [pltpu.VMEM((_B,), jnp.int32) for _ in range(3)]
        + [pltpu.VMEM((_B,), jnp.float32) for _ in range(3)]
        + [pltpu.VMEM((_B,), jnp.int32) for _ in range(3)]
        + [pltpu.VMEM((_B, _F), jnp.float32) for _ in range(3)]
        + [
            pltpu.VMEM_SHARED((_N, _F), jnp.float32),
        ]
        + [pltpu.SemaphoreType.DMA for _ in range(3)]
        + [pltpu.SemaphoreType.DMA for _ in range(3)]
        + [pltpu.SemaphoreType.DMA for _ in range(3)]
        + [pltpu.SemaphoreType.DMA]
    ),
)
def _sc_edges(y_hbm, src_hbm, dst_hbm, ew_hbm, out_hbm,
              src_0, src_1, src_2, dst_0, dst_1, dst_2,
              ew_0, ew_1, ew_2, dS_0, dS_1, dS_2,
              rw_0, rw_1, rw_2, tmp_sh,
              sA_0, sA_1, sA_2, sG_0, sG_1, sG_2,
              sS_0, sS_1, sS_2, semZ):
    src = [src_0, src_1, src_2]
    dst = [dst_0, dst_1, dst_2]
    ew = [ew_0, ew_1, ew_2]
    dS = [dS_0, dS_1, dS_2]
    rows = [rw_0, rw_1, rw_2]
    semA = [sA_0, sA_1, sA_2]
    semG = [sG_0, sG_1, sG_2]
    semS = [sS_0, sS_1, sS_2]

    cc = lax.axis_index("c")
    ss = lax.axis_index("s")

    def issue_idx(base, s):
        pltpu.async_copy(src_hbm.at[pl.ds(base, _B)], src[s], semA[s])
        pltpu.async_copy(dst_hbm.at[pl.ds(base, _B)], dst[s], semA[s])
        pltpu.async_copy(ew_hbm.at[pl.ds(base, _B)], ew[s], semA[s])

    def wait_idx(s):
        pltpu.make_async_copy(src_hbm.at[pl.ds(0, _B)], src[s], semA[s]).wait()
        pltpu.make_async_copy(dst_hbm.at[pl.ds(0, _B)], dst[s], semA[s]).wait()
        pltpu.make_async_copy(ew_hbm.at[pl.ds(0, _B)], ew[s], semA[s]).wait()

    def issue_gather(s):
        pltpu.async_copy(y_hbm.at[src[s]], rows[s], semG[s])

    def wait_gather(s):
        pltpu.make_async_copy(y_hbm.at[src[s]], rows[s], semG[s]).wait()

    def issue_scatter(s):
        pltpu.async_copy(rows[s], tmp_sh.at[dS[s]], semS[s], add=True)

    def wait_scatter(s):
        pltpu.make_async_copy(rows[s], tmp_sh.at[dS[s]], semS[s]).wait()

    def scale_rows(s):
        def grp(g, carry):
            lh = ew[s][pl.ds(g * 16, 16)]
            for i in range(16):
                sc = lh[i]
                r = g * 16 + i
                for jj in range(_F // 16):
                    rows[s][r, pl.ds(jj * 16, 16)] = (
                        rows[s][r, pl.ds(jj * 16, 16)] * sc)
            return carry

        lax.fori_loop(0, _B // 16, grp, 0)

    def cp_dstS(s):
        for g in range(_B // 16):
            dS[s][pl.ds(g * 16, 16)] = dst[s][pl.ds(g * 16, 16)]

    def _zero_rows(r, carry):
        for j in range(_F // 16):
            rw_2[r, pl.ds(j * 16, 16)] = jnp.zeros((16,), jnp.float32)
        return carry

    lax.fori_loop(0, _RB, _zero_rows, 0)

    for j in range(8):
        ci = ss + j * _NS

        @pl.when(ci < _ROW_CHUNKS)
        def _():
            pltpu.async_copy(rw_2.at[pl.ds(0, _RB)],
                             tmp_sh.at[pl.ds(ci * _RB, _RB)], semZ)

    ebase = cc * (_EP // _NC) + ss * _E_PER_TILE
    issue_idx(ebase, 0)
    issue_idx(ebase + _B, 1)
    issue_idx(ebase + 2 * _B, 2)
    wait_idx(0)
    issue_gather(0)
    wait_idx(1)
    issue_gather(1)

    for j in range(8):
        ci = ss + j * _NS

        @pl.when(ci < _ROW_CHUNKS)
        def _():
            pltpu.make_async_copy(rw_2.at[pl.ds(0, _RB)],
                                  tmp_sh.at[pl.ds(0, _RB)], semZ).wait()

    plsc.subcore_barrier()

    def chunk_step(k, s, s2, first_guard):
        wait_gather(s)
        scale_rows(s)
        cp_dstS(s)
        if first_guard is None:
            wait_scatter(s2)
        else:
            @pl.when(first_guard)
            def _():
                wait_scatter(s2)

        issue_scatter(s)

    def edge_triple(p, carry):
        for u in range(3):
            k = 3 * p + u
            s = u
            s2 = (u + 2) % 3
            chunk_step(k, s, s2, (p > 0) if u == 0 else None)
            wait_idx(s2)
            issue_gather(s2)

            @pl.when(k + 3 < _CHUNKS)
            def _():
                issue_idx(ebase + (k + 3) * _B, s)

        return carry

    lax.fori_loop(0, (_CHUNKS - 2) // 3, edge_triple, 0)
    chunk_step(_CHUNKS - 2, 0, 2, None)
    chunk_step(_CHUNKS - 1, 1, 0, None)
    wait_scatter(1)
    plsc.subcore_barrier()

    for j in range(8):
        ci = ss + j * _NS

        @pl.when(ci < _ROW_CHUNKS)
        def _():
            start = ci * _RB
            pltpu.sync_copy(tmp_sh.at[pl.ds(start, _RB)],
                            out_hbm.at[pl.ds(cc * _N + start, _RB)])


_R = 2000


def _tc_mid_body(d0_ref, d1_ref, x_ref, y_ref, dis_ref):
    d = d0_ref[...] + d1_ref[...]
    di = jnp.where(d > 0.0, lax.rsqrt(jnp.where(d > 0.0, d, 1.0)), 0.0)
    dis_ref[...] = di
    y_ref[...] = x_ref[...] * di


_tc_mid = pl.pallas_call(
    _tc_mid_body,
    grid=(_N // _R,),
    in_specs=[
        pl.BlockSpec((_R, 1), lambda i: (i, 0)),
        pl.BlockSpec((_R, 1), lambda i: (i + _N // _R, 0)),
        pl.BlockSpec((_R, _F), lambda i: (i, 0)),
    ],
    out_specs=[
        pl.BlockSpec((_R, _F), lambda i: (i, 0)),
        pl.BlockSpec((_R, 1), lambda i: (i, 0)),
    ],
    out_shape=[
        jax.ShapeDtypeStruct((_N, _F), jnp.float32),
        jax.ShapeDtypeStruct((_N, 1), jnp.float32),
    ],
)


def _tc_body(dis_ref, x_ref, t0_ref, t1_ref, w0_ref, w1_ref, bc_ref, wl_ref,
             bl_ref, out_ref):
    tx = (t0_ref[...] + t1_ref[...]) * (-dis_ref[...])
    a = (jnp.dot(x_ref[...], w0_ref[...], preferred_element_type=jnp.float32)
         + jnp.dot(tx, w1_ref[...], preferred_element_type=jnp.float32)
         + bc_ref[...])
    z = jax.nn.sigmoid(a[:, :_F])
    ht = jnp.tanh(a[:, _F:])
    h = (1.0 - z) * ht
    out_ref[...] = (jnp.dot(jnp.maximum(h, 0.0), wl_ref[...],
                            preferred_element_type=jnp.float32) + bl_ref[...])


_tc_dense = pl.pallas_call(
    _tc_body,
    grid=(_N // _R,),
    in_specs=[
        pl.BlockSpec((_R, 1), lambda i: (i, 0)),
        pl.BlockSpec((_R, _F), lambda i: (i, 0)),
        pl.BlockSpec((_R, _F), lambda i: (i, 0)),
        pl.BlockSpec((_R, _F), lambda i: (i, 0)),
        pl.BlockSpec((_F, 2 * _F), lambda i: (0, 0)),
        pl.BlockSpec((_F, 2 * _F), lambda i: (0, 0)),
        pl.BlockSpec((1, 2 * _F), lambda i: (0, 0)),
        pl.BlockSpec((_F, 1), lambda i: (0, 0)),
        pl.BlockSpec((1, 1), lambda i: (0, 0)),
    ],
    out_specs=pl.BlockSpec((_R, 1), lambda i: (i, 0)),
    out_shape=jax.ShapeDtypeStruct((_N, 1), jnp.float32),
)


def kernel(x, edge_index, edge_weight, W0_xz, W1_xz, b_xz, W0_hz, W1_hz, b_hz,
           W0_xr, W1_xr, b_xr, W0_hr, W1_hr, b_hr, W0_xh, W1_xh, b_xh,
           W0_hh, W1_hh, b_hh, W_lin, b_lin):
    pad = _EP - _E
    padidx = (jnp.arange(pad, dtype=jnp.int32) % _N).astype(jnp.int32)
    src_p = jnp.concatenate([edge_index[0], padidx])
    dst_p = jnp.concatenate([edge_index[1], padidx])
    ew_p = jnp.concatenate([edge_weight, jnp.zeros((pad,), jnp.float32)])
    deg = _sc_deg(src_p, ew_p).reshape(_NC * _N, 1)
    y, dis = _tc_mid(deg, deg, x)
    tmp = _sc_edges(y, src_p, dst_p, ew_p)
    W0c = jnp.concatenate([W0_xz, W0_xh], axis=1)
    W1c = jnp.concatenate([W1_xz, W1_xh], axis=1)
    bc = jnp.concatenate([b_xz + b_hz, b_xh + b_hh]).reshape(1, 2 * _F)
    return _tc_dense(dis, x, tmp[:_N], tmp[_N:], W0c, W1c, bc, W_lin,
                     b_lin.reshape(1, 1))
